# Initial kernel scaffold; baseline (speedup 1.0000x reference)
import jax
import jax.numpy as jnp
from jax.experimental import pallas as pl


def kernel(scores, labels, qids):
    raise NotImplementedError("write your pallas kernel here")


# speedup vs baseline: 14.4727x; 14.4727x over previous
import functools

import jax
import jax.numpy as jnp
from jax import lax
from jax.experimental import pallas as pl
from jax.experimental.pallas import tpu as pltpu
from jax.experimental.pallas import tpu_sc as plsc

_N = 1048576
_Q = 8192
_LAM = 0.5
_GAMMA = 0.2

_NC, _NS, _L = 2, 16, 16
_NW = _NC * _NS
_CHUNK = _N // _NW
_PIECE = 8192
_NPIECE = _CHUNK // _PIECE

@functools.cache
def _build_sc_kernel():
    mesh = plsc.VectorSubcoreMesh(core_axis_name="c", subcore_axis_name="s",
                                  num_cores=_NC, num_subcores=_NS)

    @functools.partial(
        pl.kernel,
        out_type=jax.ShapeDtypeStruct((_NW * 4, _Q), jnp.float32),
        mesh=mesh,
        scratch_types=[
            pltpu.VMEM((_PIECE,), jnp.float32),
            pltpu.VMEM((_PIECE,), jnp.int32),
            pltpu.VMEM((_PIECE,), jnp.int32),
            pltpu.VMEM((_Q,), jnp.float32),
            pltpu.VMEM((_Q,), jnp.float32),
            pltpu.VMEM((_Q,), jnp.float32),
            pltpu.VMEM((_Q,), jnp.float32),
        ],
        compiler_params=pltpu.CompilerParams(needs_layout_passes=False),
    )
    def _sc_segment_stats(scores_hbm, labels_hbm, qids_hbm, zeros_hbm, out_hbm,
                          s_v, l_v, q_v, ts_v, tc_v, ps_v, pc_v):
        _sc_body(scores_hbm, labels_hbm, qids_hbm, zeros_hbm, out_hbm,
                 s_v, l_v, q_v, ts_v, tc_v, ps_v, pc_v)

    return _sc_segment_stats


def _sc_body(scores_hbm, labels_hbm, qids_hbm, zeros_hbm, out_hbm,
             s_v, l_v, q_v, ts_v, tc_v, ps_v, pc_v):
    wid = lax.axis_index("s") * _NC + lax.axis_index("c")
    base = wid * _CHUNK
    pltpu.sync_copy(zeros_hbm, ts_v)
    pltpu.sync_copy(zeros_hbm, tc_v)
    pltpu.sync_copy(zeros_hbm, ps_v)
    pltpu.sync_copy(zeros_hbm, pc_v)
    ones = jnp.ones((_L,), jnp.float32)

    for p in range(_NPIECE):
        off = base + p * _PIECE
        pltpu.sync_copy(scores_hbm.at[pl.ds(off, _PIECE)], s_v)
        pltpu.sync_copy(labels_hbm.at[pl.ds(off, _PIECE)], l_v)
        pltpu.sync_copy(qids_hbm.at[pl.ds(off, _PIECE)], q_v)

        def body(i, carry):
            j = i * _L
            s = s_v[pl.ds(j, _L)]
            lf = l_v[pl.ds(j, _L)].astype(jnp.float32)
            q = q_v[pl.ds(j, _L)]
            plsc.addupdate_scatter(ts_v, [q], s)
            plsc.addupdate_scatter(tc_v, [q], ones)
            plsc.addupdate_scatter(ps_v, [q], s * lf)
            plsc.addupdate_scatter(pc_v, [q], lf)
            return carry

        lax.fori_loop(0, _PIECE // _L, body, 0)

    pltpu.sync_copy(ts_v, out_hbm.at[wid * 4 + 0])
    pltpu.sync_copy(tc_v, out_hbm.at[wid * 4 + 1])
    pltpu.sync_copy(ps_v, out_hbm.at[wid * 4 + 2])
    pltpu.sync_copy(pc_v, out_hbm.at[wid * 4 + 3])


_BLK_ROWS = 256
_G = _N // (_BLK_ROWS * 128)


def _tc_body(scores_ref, labels_ref, parts_ref, out_ref, acc_ref):
    i = pl.program_id(0)

    @pl.when(i == 0)
    def _init():
        acc_ref[0] = 0.0

    s = scores_ref[...]
    lf = labels_ref[...].astype(jnp.float32)
    log_s = jnp.maximum(jnp.log(s), -100.0)
    log_1ms = jnp.maximum(jnp.log(1.0 - s), -100.0)
    acc_ref[0] += jnp.sum(lf * log_s + (1.0 - lf) * log_1ms)

    @pl.when(i == _G - 1)
    def _finalize():
        red = jnp.sum(parts_ref[...], axis=0)
        tot_s, tot_c, pos_s, pos_c = red[0], red[1], red[2], red[3]
        neg_s = tot_s - pos_s
        neg_c = tot_c - pos_c
        valid = (pos_c > 0.0) & (neg_c > 0.0)
        pos_mean = pos_s / jnp.maximum(pos_c, 1.0)
        neg_mean = neg_s / jnp.maximum(neg_c, 1.0)
        delta = pos_mean - neg_mean
        terms = jnp.where(valid, jnp.maximum(_GAMMA - delta, 0.0), 0.0)
        n_groups = jnp.sum(valid.astype(jnp.float32))
        ldc = jnp.where(n_groups > 0.0,
                        jnp.sum(terms) / jnp.maximum(n_groups, 1.0), 0.0)
        lce = -acc_ref[0] / _N
        out_ref[0, 0] = lce + _LAM * ldc


_tc_finalize = pl.pallas_call(
    _tc_body,
    grid=(_G,),
    in_specs=[
        pl.BlockSpec((_BLK_ROWS, 128), lambda i: (i, 0)),
        pl.BlockSpec((_BLK_ROWS, 128), lambda i: (i, 0)),
        pl.BlockSpec((_NW, 4, _Q // 128, 128), lambda i: (0, 0, 0, 0)),
    ],
    out_specs=pl.BlockSpec(memory_space=pltpu.SMEM),
    out_shape=jax.ShapeDtypeStruct((1, 1), jnp.float32),
    scratch_shapes=[pltpu.SMEM((1,), jnp.float32)],
)


def kernel(scores, labels, qids):
    labels_i = labels.astype(jnp.int32)
    qids_i = qids.astype(jnp.int32)
    zeros = jnp.zeros((_Q,), jnp.float32)
    parts = _build_sc_kernel()(scores, labels_i, qids_i, zeros)
    s2 = scores.reshape(_N // 128, 128)
    l2 = labels_i.reshape(_N // 128, 128)
    p4 = parts.reshape(_NW, 4, _Q // 128, 128)
    out = _tc_finalize(s2, l2, p4)
    return out[0, 0]


# speedup vs baseline: 31.1162x; 2.1500x over previous
import functools

import jax
import jax.numpy as jnp
from jax import lax
from jax.experimental import pallas as pl
from jax.experimental.pallas import tpu as pltpu
from jax.experimental.pallas import tpu_sc as plsc

_N = 1048576
_Q = 8192
_LAM = 0.5
_GAMMA = 0.2

_NC, _NS, _L = 2, 16, 16
_NW = _NC * _NS
_CHUNK = _N // _NW
_PIECE = 8192
_NPIECE = _CHUNK // _PIECE

@functools.cache
def _build_sc_kernel():
    mesh = plsc.VectorSubcoreMesh(core_axis_name="c", subcore_axis_name="s",
                                  num_cores=_NC, num_subcores=_NS)

    @functools.partial(
        pl.kernel,
        out_type=jax.ShapeDtypeStruct((_NW * 4, _Q), jnp.float32),
        mesh=mesh,
        scratch_types=[
            pltpu.VMEM((_PIECE,), jnp.float32),
            pltpu.VMEM((_PIECE,), jnp.int32),
            pltpu.VMEM((_PIECE + _L,), jnp.int32),
            pltpu.VMEM((_Q,), jnp.float32),
            pltpu.VMEM((_Q,), jnp.float32),
            pltpu.VMEM((_Q,), jnp.float32),
            pltpu.VMEM((_Q,), jnp.float32),
        ],
        compiler_params=pltpu.CompilerParams(needs_layout_passes=False),
    )
    def _sc_segment_stats(scores_hbm, labels_hbm, qids_hbm, zeros_hbm, out_hbm,
                          s_v, l_v, q_v, ts_v, tc_v, ps_v, pc_v):
        _sc_body(scores_hbm, labels_hbm, qids_hbm, zeros_hbm, out_hbm,
                 s_v, l_v, q_v, ts_v, tc_v, ps_v, pc_v)

    return _sc_segment_stats


def _sc_body(scores_hbm, labels_hbm, qids_hbm, zeros_hbm, out_hbm,
             s_v, l_v, q_v, ts_v, tc_v, ps_v, pc_v):
    wid = lax.axis_index("s") * _NC + lax.axis_index("c")
    base = wid * _CHUNK
    pltpu.sync_copy(zeros_hbm, ts_v)
    pltpu.sync_copy(zeros_hbm, tc_v)
    pltpu.sync_copy(zeros_hbm, ps_v)
    pltpu.sync_copy(zeros_hbm, pc_v)

    lane = lax.iota(jnp.int32, _L)
    last_lane = lane == (_L - 1)
    not_last = lane != (_L - 1)
    c_one = (lane + 1).astype(jnp.float32)
    neg_c_one = -c_one

    for p in range(_NPIECE):
        off = base + p * _PIECE
        pltpu.sync_copy(scores_hbm.at[pl.ds(off, _PIECE)], s_v)
        pltpu.sync_copy(labels_hbm.at[pl.ds(off, _PIECE)], l_v)
        pltpu.sync_copy(qids_hbm.at[pl.ds(off, _PIECE)],
                        q_v.at[pl.ds(0, _PIECE)])

        def body(i, carry):
            j = i * _L
            s = s_v[pl.ds(j, _L)]
            lf = l_v[pl.ds(j, _L)].astype(jnp.float32)
            q = q_v[pl.ds(j, _L)]
            qn = q_v[pl.ds(j + 1, _L)]
            boundary = q != qn
            flush = boundary | last_lane
            bsub = boundary & not_last
            c_s = plsc.cumsum(s)
            c_sl = plsc.cumsum(s * lf)
            c_lf = plsc.cumsum(lf)
            plsc.addupdate_scatter(ts_v, [q], c_s, mask=flush)
            plsc.addupdate_scatter(tc_v, [q], c_one, mask=flush)
            plsc.addupdate_scatter(ps_v, [q], c_sl, mask=flush)
            plsc.addupdate_scatter(pc_v, [q], c_lf, mask=flush)
            plsc.addupdate_scatter(ts_v, [qn], -c_s, mask=bsub)
            plsc.addupdate_scatter(tc_v, [qn], neg_c_one, mask=bsub)
            plsc.addupdate_scatter(ps_v, [qn], -c_sl, mask=bsub)
            plsc.addupdate_scatter(pc_v, [qn], -c_lf, mask=bsub)
            return carry

        lax.fori_loop(0, _PIECE // _L, body, 0)

    pltpu.sync_copy(ts_v, out_hbm.at[wid * 4 + 0])
    pltpu.sync_copy(tc_v, out_hbm.at[wid * 4 + 1])
    pltpu.sync_copy(ps_v, out_hbm.at[wid * 4 + 2])
    pltpu.sync_copy(pc_v, out_hbm.at[wid * 4 + 3])


_BLK_ROWS = 256
_G = _N // (_BLK_ROWS * 128)


def _tc_body(scores_ref, labels_ref, parts_ref, out_ref, acc_ref):
    i = pl.program_id(0)

    @pl.when(i == 0)
    def _init():
        acc_ref[0] = 0.0

    s = scores_ref[...]
    lf = labels_ref[...].astype(jnp.float32)
    log_s = jnp.maximum(jnp.log(s), -100.0)
    log_1ms = jnp.maximum(jnp.log(1.0 - s), -100.0)
    acc_ref[0] += jnp.sum(lf * log_s + (1.0 - lf) * log_1ms)

    @pl.when(i == _G - 1)
    def _finalize():
        red = jnp.sum(parts_ref[...], axis=0)
        tot_s, tot_c, pos_s, pos_c = red[0], red[1], red[2], red[3]
        neg_s = tot_s - pos_s
        neg_c = tot_c - pos_c
        valid = (pos_c > 0.0) & (neg_c > 0.0)
        pos_mean = pos_s / jnp.maximum(pos_c, 1.0)
        neg_mean = neg_s / jnp.maximum(neg_c, 1.0)
        delta = pos_mean - neg_mean
        terms = jnp.where(valid, jnp.maximum(_GAMMA - delta, 0.0), 0.0)
        n_groups = jnp.sum(valid.astype(jnp.float32))
        ldc = jnp.where(n_groups > 0.0,
                        jnp.sum(terms) / jnp.maximum(n_groups, 1.0), 0.0)
        lce = -acc_ref[0] / _N
        out_ref[0, 0] = lce + _LAM * ldc


_tc_finalize = pl.pallas_call(
    _tc_body,
    grid=(_G,),
    in_specs=[
        pl.BlockSpec((_BLK_ROWS, 128), lambda i: (i, 0)),
        pl.BlockSpec((_BLK_ROWS, 128), lambda i: (i, 0)),
        pl.BlockSpec((_NW, 4, _Q // 128, 128), lambda i: (0, 0, 0, 0)),
    ],
    out_specs=pl.BlockSpec(memory_space=pltpu.SMEM),
    out_shape=jax.ShapeDtypeStruct((1, 1), jnp.float32),
    scratch_shapes=[pltpu.SMEM((1,), jnp.float32)],
)


def kernel(scores, labels, qids):
    labels_i = labels.astype(jnp.int32)
    qids_i = qids.astype(jnp.int32)
    zeros = jnp.zeros((_Q,), jnp.float32)
    parts = _build_sc_kernel()(scores, labels_i, qids_i, zeros)
    s2 = scores.reshape(_N // 128, 128)
    l2 = labels_i.reshape(_N // 128, 128)
    p4 = parts.reshape(_NW, 4, _Q // 128, 128)
    out = _tc_finalize(s2, l2, p4)
    return out[0, 0]


# speedup vs baseline: 56.9532x; 1.8303x over previous
import functools

import jax
import jax.numpy as jnp
from jax import lax
from jax.experimental import pallas as pl
from jax.experimental.pallas import tpu as pltpu
from jax.experimental.pallas import tpu_sc as plsc

_N = 1048576
_Q = 8192
_LAM = 0.5
_GAMMA = 0.2

_NC, _NS, _L = 2, 16, 16
_NW = _NC * _NS
_CHUNK = _N // _NW
_PIECE = 8192
_NPIECE = _CHUNK // _PIECE


@functools.cache
def _build_sc_kernel():
    mesh = plsc.VectorSubcoreMesh(core_axis_name="c", subcore_axis_name="s",
                                  num_cores=_NC, num_subcores=_NS)

    @functools.partial(
        pl.kernel,
        out_type=jax.ShapeDtypeStruct((4, _NW, _Q), jnp.float32),
        mesh=mesh,
        scratch_types=[
            pltpu.VMEM((_PIECE,), jnp.float32),
            pltpu.VMEM((_PIECE,), jnp.int32),
            pltpu.VMEM((_PIECE + _L,), jnp.int32),
            pltpu.VMEM((_Q,), jnp.float32),
            pltpu.VMEM((_Q,), jnp.float32),
            pltpu.VMEM((_Q,), jnp.float32),
            pltpu.VMEM((_Q,), jnp.float32),
        ],
        compiler_params=pltpu.CompilerParams(needs_layout_passes=False),
    )
    def _sc_segment_stats(scores_hbm, labels_hbm, qids_hbm, out_hbm,
                          s_v, l_v, q_v, ts_v, tc_v, ps_v, pc_v):
        _sc_body(scores_hbm, labels_hbm, qids_hbm, out_hbm,
                 s_v, l_v, q_v, ts_v, tc_v, ps_v, pc_v)

    return _sc_segment_stats


def _sc_body(scores_hbm, labels_hbm, qids_hbm, out_hbm,
             s_v, l_v, q_v, ts_v, tc_v, ps_v, pc_v):
    wid = lax.axis_index("s") * _NC + lax.axis_index("c")
    base = wid * _CHUNK

    zero = jnp.zeros((_L,), jnp.float32)

    @plsc.parallel_loop(0, _Q // _L)
    def _zero(i):
        j = i * _L
        ts_v[pl.ds(j, _L)] = zero
        tc_v[pl.ds(j, _L)] = zero
        ps_v[pl.ds(j, _L)] = zero
        pc_v[pl.ds(j, _L)] = zero

    lane = lax.iota(jnp.int32, _L)
    last_lane = lane == (_L - 1)
    not_last = lane != (_L - 1)
    c_one = (lane + 1).astype(jnp.float32)
    neg_c_one = -c_one

    for p in range(_NPIECE):
        off = base + p * _PIECE
        pltpu.sync_copy(scores_hbm.at[pl.ds(off, _PIECE)], s_v)
        pltpu.sync_copy(labels_hbm.at[pl.ds(off, _PIECE)], l_v)
        pltpu.sync_copy(qids_hbm.at[pl.ds(off, _PIECE)],
                        q_v.at[pl.ds(0, _PIECE)])

        @plsc.parallel_loop(0, _PIECE // _L, unroll=4)
        def _body(i):
            j = i * _L
            s = s_v[pl.ds(j, _L)]
            lf = l_v[pl.ds(j, _L)].astype(jnp.float32)
            q = q_v[pl.ds(j, _L)]
            qn = q_v[pl.ds(j + 1, _L)]
            boundary = q != qn
            flush = boundary | last_lane
            bsub = boundary & not_last
            c_s = plsc.cumsum(s)
            c_sl = plsc.cumsum(s * lf)
            c_lf = plsc.cumsum(lf)
            plsc.addupdate_scatter(ts_v, [q], c_s, mask=flush)
            plsc.addupdate_scatter(tc_v, [q], c_one, mask=flush)
            plsc.addupdate_scatter(ps_v, [q], c_sl, mask=flush)
            plsc.addupdate_scatter(pc_v, [q], c_lf, mask=flush)
            plsc.addupdate_scatter(ts_v, [qn], -c_s, mask=bsub)
            plsc.addupdate_scatter(tc_v, [qn], neg_c_one, mask=bsub)
            plsc.addupdate_scatter(ps_v, [qn], -c_sl, mask=bsub)
            plsc.addupdate_scatter(pc_v, [qn], -c_lf, mask=bsub)

    pltpu.sync_copy(ts_v, out_hbm.at[0, wid])
    pltpu.sync_copy(tc_v, out_hbm.at[1, wid])
    pltpu.sync_copy(ps_v, out_hbm.at[2, wid])
    pltpu.sync_copy(pc_v, out_hbm.at[3, wid])


_BLK_ROWS = 256
_G = _N // (_BLK_ROWS * 128)


def _tc_bce_body(scores_ref, labels_ref, out_ref, acc_ref):
    i = pl.program_id(0)

    @pl.when(i == 0)
    def _init():
        acc_ref[0] = 0.0

    s = scores_ref[...]
    lf = labels_ref[...].astype(jnp.float32)
    log_s = jnp.maximum(jnp.log(s), -100.0)
    log_1ms = jnp.maximum(jnp.log(1.0 - s), -100.0)
    acc_ref[0] += jnp.sum(lf * log_s + (1.0 - lf) * log_1ms)

    @pl.when(i == _G - 1)
    def _done():
        out_ref[0, 0] = acc_ref[0]


_tc_bce = pl.pallas_call(
    _tc_bce_body,
    grid=(_G,),
    in_specs=[
        pl.BlockSpec((_BLK_ROWS, 128), lambda i: (i, 0)),
        pl.BlockSpec((_BLK_ROWS, 128), lambda i: (i, 0)),
    ],
    out_specs=pl.BlockSpec(memory_space=pltpu.SMEM),
    out_shape=jax.ShapeDtypeStruct((1, 1), jnp.float32),
    scratch_shapes=[pltpu.SMEM((1,), jnp.float32)],
)


def _tc_fin_body(parts_ref, bce_ref, out_ref):
    red = jnp.sum(parts_ref[...], axis=1)
    tot_s = red[0:1, :]
    tot_c = red[1:2, :]
    pos_s = red[2:3, :]
    pos_c = red[3:4, :]
    neg_s = tot_s - pos_s
    neg_c = tot_c - pos_c
    valid = (pos_c > 0.0) & (neg_c > 0.0)
    pos_mean = pos_s / jnp.maximum(pos_c, 1.0)
    neg_mean = neg_s / jnp.maximum(neg_c, 1.0)
    delta = pos_mean - neg_mean
    terms = jnp.where(valid, jnp.maximum(_GAMMA - delta, 0.0), 0.0)
    n_groups = jnp.sum(valid.astype(jnp.float32))
    ldc = jnp.where(n_groups > 0.0,
                    jnp.sum(terms) / jnp.maximum(n_groups, 1.0), 0.0)
    lce = -bce_ref[0, 0] / _N
    out_ref[0, 0] = lce + _LAM * ldc


_tc_finalize = pl.pallas_call(
    _tc_fin_body,
    in_specs=[
        pl.BlockSpec(memory_space=pltpu.VMEM),
        pl.BlockSpec(memory_space=pltpu.SMEM),
    ],
    out_specs=pl.BlockSpec(memory_space=pltpu.SMEM),
    out_shape=jax.ShapeDtypeStruct((1, 1), jnp.float32),
)


def kernel(scores, labels, qids):
    labels_i = labels.astype(jnp.int32)
    qids_i = qids.astype(jnp.int32)
    parts = _build_sc_kernel()(scores, labels_i, qids_i)
    s2 = scores.reshape(_N // 128, 128)
    l2 = labels_i.reshape(_N // 128, 128)
    bce = _tc_bce(s2, l2)
    out = _tc_finalize(parts, bce)
    return out[0, 0]


# speedup vs baseline: 71.4057x; 1.2538x over previous
import functools

import jax
import jax.numpy as jnp
from jax import lax
from jax.experimental import pallas as pl
from jax.experimental.pallas import tpu as pltpu
from jax.experimental.pallas import tpu_sc as plsc

_N = 1048576
_Q = 8192
_LAM = 0.5
_GAMMA = 0.2

_NC, _NS, _L = 2, 16, 16
_NW = _NC * _NS
_CHUNK = _N // _NW
_PIECE = 8192
_NPIECE = _CHUNK // _PIECE


@functools.cache
def _build_sc_kernel():
    mesh = plsc.VectorSubcoreMesh(core_axis_name="c", subcore_axis_name="s",
                                  num_cores=_NC, num_subcores=_NS)

    @functools.partial(
        pl.kernel,
        out_type=jax.ShapeDtypeStruct((4, _NW, _Q), jnp.float32),
        mesh=mesh,
        scratch_types=[
            pltpu.VMEM((_PIECE,), jnp.float32),
            pltpu.VMEM((_PIECE,), jnp.float32),
            pltpu.VMEM((_PIECE,), jnp.int32),
            pltpu.VMEM((_PIECE,), jnp.int32),
            pltpu.VMEM((_PIECE + _L,), jnp.int32),
            pltpu.VMEM((_PIECE + _L,), jnp.int32),
            pltpu.VMEM((_Q,), jnp.float32),
            pltpu.VMEM((_Q,), jnp.float32),
            pltpu.VMEM((_Q,), jnp.float32),
            pltpu.VMEM((_Q,), jnp.float32),
            pltpu.SemaphoreType.DMA,
            pltpu.SemaphoreType.DMA,
        ],
        compiler_params=pltpu.CompilerParams(needs_layout_passes=False),
    )
    def _sc_segment_stats(scores_hbm, labels_hbm, qids_hbm, out_hbm,
                          s0_v, s1_v, l0_v, l1_v, q0_v, q1_v,
                          ts_v, tc_v, ps_v, pc_v, sem0, sem1):
        _sc_body(scores_hbm, labels_hbm, qids_hbm, out_hbm,
                 (s0_v, s1_v), (l0_v, l1_v), (q0_v, q1_v),
                 ts_v, tc_v, ps_v, pc_v, (sem0, sem1))

    return _sc_segment_stats


def _sc_body(scores_hbm, labels_hbm, qids_hbm, out_hbm,
             s_bufs, l_bufs, q_bufs, ts_v, tc_v, ps_v, pc_v, sems):
    wid = lax.axis_index("s") * _NC + lax.axis_index("c")
    base = wid * _CHUNK

    def start_piece(p):
        b = p % 2
        off = base + p * _PIECE
        sem = sems[b]
        return (
            pltpu.async_copy(scores_hbm.at[pl.ds(off, _PIECE)],
                             s_bufs[b], sem),
            pltpu.async_copy(labels_hbm.at[pl.ds(off, _PIECE)],
                             l_bufs[b], sem),
            pltpu.async_copy(qids_hbm.at[pl.ds(off, _PIECE)],
                             q_bufs[b].at[pl.ds(0, _PIECE)], sem),
        )

    handles = {0: start_piece(0), 1: start_piece(1)}

    zero = jnp.zeros((_L,), jnp.float32)

    @plsc.parallel_loop(0, _Q // _L)
    def _zero(i):
        j = i * _L
        ts_v[pl.ds(j, _L)] = zero
        tc_v[pl.ds(j, _L)] = zero
        ps_v[pl.ds(j, _L)] = zero
        pc_v[pl.ds(j, _L)] = zero

    lane = lax.iota(jnp.int32, _L)
    last_lane = lane == (_L - 1)
    not_last = lane != (_L - 1)
    c_one = (lane + 1).astype(jnp.float32)
    neg_c_one = -c_one

    for p in range(_NPIECE):
        b = p % 2
        for h in handles.pop(p):
            h.wait()
        if p + 2 < _NPIECE:
            handles[p + 2] = start_piece(p + 2)
        sb_v = s_bufs[b]
        lb_v = l_bufs[b]
        qb_v = q_bufs[b]

        @plsc.parallel_loop(0, _PIECE // _L, unroll=8)
        def _body(i):
            j = i * _L
            s = sb_v[pl.ds(j, _L)]
            lf = lb_v[pl.ds(j, _L)].astype(jnp.float32)
            q = qb_v[pl.ds(j, _L)]
            qn = qb_v[pl.ds(j + 1, _L)]
            boundary = q != qn
            flush = boundary | last_lane
            bsub = boundary & not_last
            c_s = plsc.cumsum(s)
            c_sl = plsc.cumsum(s * lf)
            c_lf = plsc.cumsum(lf)
            plsc.addupdate_scatter(ts_v, [q], c_s, mask=flush)
            plsc.addupdate_scatter(tc_v, [q], c_one, mask=flush)
            plsc.addupdate_scatter(ps_v, [q], c_sl, mask=flush)
            plsc.addupdate_scatter(pc_v, [q], c_lf, mask=flush)
            plsc.addupdate_scatter(ts_v, [qn], -c_s, mask=bsub)
            plsc.addupdate_scatter(tc_v, [qn], neg_c_one, mask=bsub)
            plsc.addupdate_scatter(ps_v, [qn], -c_sl, mask=bsub)
            plsc.addupdate_scatter(pc_v, [qn], -c_lf, mask=bsub)

    pltpu.sync_copy(ts_v, out_hbm.at[0, wid])
    pltpu.sync_copy(tc_v, out_hbm.at[1, wid])
    pltpu.sync_copy(ps_v, out_hbm.at[2, wid])
    pltpu.sync_copy(pc_v, out_hbm.at[3, wid])


_BLK_ROWS = 256
_G = _N // (_BLK_ROWS * 128)


def _tc_bce_body(scores_ref, labels_ref, out_ref, acc_ref):
    i = pl.program_id(0)

    @pl.when(i == 0)
    def _init():
        acc_ref[0] = 0.0

    s = scores_ref[...]
    pos = labels_ref[...] == 1
    t = jnp.where(pos, s, 1.0 - s)
    acc_ref[0] += jnp.sum(jnp.maximum(jnp.log(t), -100.0))

    @pl.when(i == _G - 1)
    def _done():
        out_ref[0, 0] = acc_ref[0]


_tc_bce = pl.pallas_call(
    _tc_bce_body,
    grid=(_G,),
    in_specs=[
        pl.BlockSpec((_BLK_ROWS, 128), lambda i: (i, 0)),
        pl.BlockSpec((_BLK_ROWS, 128), lambda i: (i, 0)),
    ],
    out_specs=pl.BlockSpec(memory_space=pltpu.SMEM),
    out_shape=jax.ShapeDtypeStruct((1, 1), jnp.float32),
    scratch_shapes=[pltpu.SMEM((1,), jnp.float32)],
)


def _tc_fin_body(parts_ref, bce_ref, out_ref):
    red = jnp.sum(parts_ref[...], axis=1)
    tot_s = red[0:1, :]
    tot_c = red[1:2, :]
    pos_s = red[2:3, :]
    pos_c = red[3:4, :]
    neg_s = tot_s - pos_s
    neg_c = tot_c - pos_c
    valid = (pos_c > 0.0) & (neg_c > 0.0)
    pos_mean = pos_s / jnp.maximum(pos_c, 1.0)
    neg_mean = neg_s / jnp.maximum(neg_c, 1.0)
    delta = pos_mean - neg_mean
    terms = jnp.where(valid, jnp.maximum(_GAMMA - delta, 0.0), 0.0)
    n_groups = jnp.sum(valid.astype(jnp.float32))
    ldc = jnp.where(n_groups > 0.0,
                    jnp.sum(terms) / jnp.maximum(n_groups, 1.0), 0.0)
    lce = -bce_ref[0, 0] / _N
    out_ref[0, 0] = lce + _LAM * ldc


_tc_finalize = pl.pallas_call(
    _tc_fin_body,
    in_specs=[
        pl.BlockSpec(memory_space=pltpu.VMEM),
        pl.BlockSpec(memory_space=pltpu.SMEM),
    ],
    out_specs=pl.BlockSpec(memory_space=pltpu.SMEM),
    out_shape=jax.ShapeDtypeStruct((1, 1), jnp.float32),
)


def kernel(scores, labels, qids):
    labels_i = labels.astype(jnp.int32)
    qids_i = qids.astype(jnp.int32)
    parts = _build_sc_kernel()(scores, labels_i, qids_i)
    s2 = scores.reshape(_N // 128, 128)
    l2 = labels_i.reshape(_N // 128, 128)
    bce = _tc_bce(s2, l2)
    out = _tc_finalize(parts, bce)
    return out[0, 0]
